# baseline (speedup 1.0000x reference)
import functools

import jax
import jax.numpy as jnp
from jax import lax
from jax.experimental import pallas as pl
from jax.experimental.pallas import tpu as pltpu
from jax.experimental.pallas import tpu_sc as plsc

VOCAB = 1000000
EMBED_DIM = 32
BATCH = 4096
MAX_LEN = 200

_INFO = plsc.get_sparse_core_info()
_NC = _INFO.num_cores
_NS = _INFO.num_subcores
_NW = _NC * _NS

_B_PER_W = BATCH // _NW
_HALF = MAX_LEN // 2
_CHUNK_B = 4
_NSTR = _CHUNK_B * 2
_STEPS = _B_PER_W // _CHUNK_B
_PAIRS = _STEPS // 2
_IDX_ROWS = _B_PER_W * 2


def _make_kernel():
    mesh = plsc.VectorSubcoreMesh(core_axis_name="c", subcore_axis_name="s")

    @functools.partial(
        pl.kernel,
        mesh=mesh,
        compiler_params=pltpu.CompilerParams(use_tc_tiling_on_sc=False),
        out_type=jax.ShapeDtypeStruct((BATCH, MAX_LEN, EMBED_DIM), jnp.float32),
        scratch_types=[
            pltpu.VMEM((_IDX_ROWS, _HALF), jnp.int32),
            pltpu.VMEM((_CHUNK_B, MAX_LEN, EMBED_DIM), jnp.float32),
            pltpu.VMEM((_CHUNK_B, MAX_LEN, EMBED_DIM), jnp.float32),
            pltpu.SemaphoreType.DMA,
            pltpu.SemaphoreType.DMA,
            pltpu.SemaphoreType.DMA,
            pltpu.SemaphoreType.DMA,
        ],
    )
    def emb_kernel(idx_hbm, table_hbm, out_hbm, idx_v, rows0, rows1,
                   sg0, sg1, ss0, ss1):
        wid = lax.axis_index("s") * _NC + lax.axis_index("c")
        pltpu.sync_copy(idx_hbm.at[wid], idx_v)
        b_base = wid * _B_PER_W

        def fire(c, buf, sem):
            for j in range(_NSTR):
                pltpu.async_copy(
                    table_hbm.at[idx_v.at[c * _NSTR + j]],
                    buf.at[j // 2, pl.ds((j % 2) * _HALF, _HALF), :],
                    sem,
                )

        def drain_gather(buf, sem):
            pltpu.make_async_copy(out_hbm.at[pl.ds(0, _CHUNK_B)], buf, sem).wait()

        def store_start(buf, c, sem):
            pltpu.async_copy(
                buf, out_hbm.at[pl.ds(b_base + c * _CHUNK_B, _CHUNK_B)], sem
            )

        def store_wait(buf, sem):
            pltpu.make_async_copy(buf, out_hbm.at[pl.ds(0, _CHUNK_B)], sem).wait()

        fire(0, rows0, sg0)
        fire(1, rows1, sg1)

        def pair(p, _):
            c0 = 2 * p
            drain_gather(rows0, sg0)
            store_start(rows0, c0, ss0)
            drain_gather(rows1, sg1)
            store_start(rows1, c0 + 1, ss1)
            store_wait(rows0, ss0)
            fire(c0 + 2, rows0, sg0)
            store_wait(rows1, ss1)
            fire(c0 + 3, rows1, sg1)
            return 0

        lax.fori_loop(0, _PAIRS - 1, pair, 0)

        c0 = _STEPS - 2
        drain_gather(rows0, sg0)
        store_start(rows0, c0, ss0)
        drain_gather(rows1, sg1)
        store_start(rows1, c0 + 1, ss1)
        store_wait(rows0, ss0)
        store_wait(rows1, ss1)

    return emb_kernel


_EMB_KERNEL = _make_kernel()


@jax.jit
def kernel(inputs, table):
    idx = inputs.astype(jnp.int32).reshape(_NW, _IDX_ROWS, _HALF)
    return _EMB_KERNEL(idx, table)
